# Initial kernel scaffold; baseline (speedup 1.0000x reference)
import jax
import jax.numpy as jnp
from jax.experimental import pallas as pl


def kernel(x, edge_index, W, b):
    raise NotImplementedError("write your pallas kernel here")


# speedup vs baseline: 12.9756x; 12.9756x over previous
import functools

import jax
import jax.numpy as jnp
from jax import lax
from jax.experimental import pallas as pl
from jax.experimental.pallas import tpu as pltpu
from jax.experimental.pallas import tpu_sc as plsc

N = 10000
E = 160000
D = 256
DH = 128
EBLK = 128
NBLKS = E // EBLK
NC, NS = 2, 16
HIST_N = 10240
HSLC = HIST_N // NS
RPT = 632
RPT_LAST = N - (NS - 1) * RPT
R = 1000


def _sc_mesh():
    return plsc.VectorSubcoreMesh(core_axis_name="c", subcore_axis_name="s")


def _deg_body(edge_hbm, deg0_hbm, deg1_hbm, dst_v, ones_v, zeros_v, hist_sh):
    c = lax.axis_index("c")
    s = lax.axis_index("s")

    for j in range(EBLK // 16):
        ones_v[pl.ds(j * 16, 16)] = jnp.ones((16,), jnp.float32)
    for j in range(HSLC // 16):
        zeros_v[pl.ds(j * 16, 16)] = jnp.zeros((16,), jnp.float32)

    pltpu.sync_copy(zeros_v, hist_sh.at[pl.ds(s * HSLC, HSLC)])
    plsc.subcore_barrier()

    w = c * NS + s

    @pl.loop(0, (NBLKS + NC * NS - 1) // (NC * NS))
    def _edge_blocks(i):
        bi = w + i * NC * NS

        @pl.when(bi < NBLKS)
        def _():
            pltpu.sync_copy(edge_hbm.at[1, pl.ds(bi * EBLK, EBLK)], dst_v)
            pltpu.sync_copy(ones_v, hist_sh.at[dst_v], add=True)

    plsc.subcore_barrier()

    @pl.when(c == 0)
    def _():
        pltpu.sync_copy(hist_sh.at[pl.ds(s * HSLC, HSLC)],
                        deg0_hbm.at[pl.ds(s * HSLC, HSLC)])

    @pl.when(c == 1)
    def _():
        pltpu.sync_copy(hist_sh.at[pl.ds(s * HSLC, HSLC)],
                        deg1_hbm.at[pl.ds(s * HSLC, HSLC)])


_deg_call = pl.kernel(
    _deg_body,
    out_type=(jax.ShapeDtypeStruct((HIST_N,), jnp.float32),
              jax.ShapeDtypeStruct((HIST_N,), jnp.float32)),
    mesh=_sc_mesh(),
    scratch_types=[
        pltpu.VMEM((EBLK,), jnp.int32),
        pltpu.VMEM((EBLK,), jnp.float32),
        pltpu.VMEM((HSLC,), jnp.float32),
        pltpu.VMEM_SHARED((HIST_N,), jnp.float32),
    ],
)


def _mm_body(d0_ref, d1_ref, x_ref, w_ref, g0_ref, g1_ref):
    deg = d0_ref[...] + d1_ref[...] + 1.0
    dinv = lax.rsqrt(deg)
    h = jnp.dot(x_ref[...], w_ref[...], preferred_element_type=jnp.float32)
    g = h * dinv
    g0_ref[...] = g[:, :DH]
    g1_ref[...] = g[:, DH:]


_mm_call = pl.pallas_call(
    _mm_body,
    grid=(N // R,),
    in_specs=[
        pl.BlockSpec((R, 1), lambda i: (i, 0)),
        pl.BlockSpec((R, 1), lambda i: (i, 0)),
        pl.BlockSpec((R, D), lambda i: (i, 0)),
        pl.BlockSpec((D, D), lambda i: (0, 0)),
    ],
    out_specs=[
        pl.BlockSpec((R, DH), lambda i: (i, 0)),
        pl.BlockSpec((R, DH), lambda i: (i, 0)),
    ],
    out_shape=[
        jax.ShapeDtypeStruct((N, DH), jnp.float32),
        jax.ShapeDtypeStruct((N, DH), jnp.float32),
    ],
)


def _edge_body(g0_hbm, g1_hbm, edge_hbm, a0_hbm, a1_hbm,
               src_v, dst_v, rows_v, sem, acc_sh):
    c = lax.axis_index("c")
    s = lax.axis_index("s")

    def work(g_hbm, o_hbm):
        @pl.when(s < NS - 1)
        def _():
            pltpu.sync_copy(g_hbm.at[pl.ds(s * RPT, RPT)],
                            acc_sh.at[pl.ds(s * RPT, RPT)])

        @pl.when(s == NS - 1)
        def _():
            pltpu.sync_copy(g_hbm.at[pl.ds((NS - 1) * RPT, RPT_LAST)],
                            acc_sh.at[pl.ds((NS - 1) * RPT, RPT_LAST)])

        plsc.subcore_barrier()

        @pl.loop(0, (NBLKS + NS - 1) // NS)
        def _blocks(i):
            bi = s + i * NS

            @pl.when(bi < NBLKS)
            def _():
                off = bi * EBLK
                pltpu.sync_copy(edge_hbm.at[0, pl.ds(off, EBLK)], src_v)
                pltpu.sync_copy(edge_hbm.at[1, pl.ds(off, EBLK)], dst_v)
                pltpu.async_copy(g_hbm.at[src_v], rows_v, sem).wait()
                pltpu.sync_copy(rows_v, acc_sh.at[dst_v], add=True)

        plsc.subcore_barrier()

        @pl.when(s < NS - 1)
        def _():
            pltpu.sync_copy(acc_sh.at[pl.ds(s * RPT, RPT)],
                            o_hbm.at[pl.ds(s * RPT, RPT)])

        @pl.when(s == NS - 1)
        def _():
            pltpu.sync_copy(acc_sh.at[pl.ds((NS - 1) * RPT, RPT_LAST)],
                            o_hbm.at[pl.ds((NS - 1) * RPT, RPT_LAST)])

    @pl.when(c == 0)
    def _():
        work(g0_hbm, a0_hbm)

    @pl.when(c == 1)
    def _():
        work(g1_hbm, a1_hbm)


_edge_call = pl.kernel(
    _edge_body,
    out_type=(jax.ShapeDtypeStruct((N, DH), jnp.float32),
              jax.ShapeDtypeStruct((N, DH), jnp.float32)),
    mesh=_sc_mesh(),
    scratch_types=[
        pltpu.VMEM((EBLK,), jnp.int32),
        pltpu.VMEM((EBLK,), jnp.int32),
        pltpu.VMEM((EBLK, DH), jnp.float32),
        pltpu.SemaphoreType.DMA,
        pltpu.VMEM_SHARED((N, DH), jnp.float32),
    ],
)


def _ep_body(d0_ref, d1_ref, b_ref, a0_ref, a1_ref, o_ref):
    dinv = lax.rsqrt(d0_ref[...] + d1_ref[...] + 1.0)
    o_ref[:, :DH] = a0_ref[...] * dinv + b_ref[:, :DH]
    o_ref[:, DH:] = a1_ref[...] * dinv + b_ref[:, DH:]


_ep_call = pl.pallas_call(
    _ep_body,
    grid=(N // R,),
    in_specs=[
        pl.BlockSpec((R, 1), lambda i: (i, 0)),
        pl.BlockSpec((R, 1), lambda i: (i, 0)),
        pl.BlockSpec((1, D), lambda i: (0, 0)),
        pl.BlockSpec((R, DH), lambda i: (i, 0)),
        pl.BlockSpec((R, DH), lambda i: (i, 0)),
    ],
    out_specs=pl.BlockSpec((R, D), lambda i: (i, 0)),
    out_shape=jax.ShapeDtypeStruct((N, D), jnp.float32),
)


def kernel(x, edge_index, W, b):
    deg0, deg1 = _deg_call(edge_index)
    d0 = deg0[:N].reshape(N, 1)
    d1 = deg1[:N].reshape(N, 1)
    g0, g1 = _mm_call(d0, d1, x, W)
    a0, a1 = _edge_call(g0, g1, edge_index)
    return _ep_call(d0, d1, b.reshape(1, D), a0, a1)
